# baseline (speedup 1.0000x reference)
import functools

import jax
import jax.numpy as jnp
from jax import lax
from jax.experimental import pallas as pl
from jax.experimental.pallas import tpu as pltpu
from jax.experimental.pallas import tpu_sc as plsc

GAMMA = 12.0
HIDDEN = 64
BATCH = 16384
LANES = 16
NUM_WORKERS = 32
B_PER_W = BATCH // NUM_WORKERS
N_CHUNKS = 4
CHUNK = B_PER_W // N_CHUNKS


def _make_kernel():
  mesh = plsc.VectorSubcoreMesh(core_axis_name="c", subcore_axis_name="s")

  @functools.partial(
      pl.kernel,
      mesh=mesh,
      compiler_params=pltpu.CompilerParams(
          needs_layout_passes=False, use_tc_tiling_on_sc=False),
      out_type=jax.ShapeDtypeStruct((NUM_WORKERS, B_PER_W), jnp.float32),
      scratch_types=[
          pltpu.VMEM((N_CHUNKS, CHUNK), jnp.int32),
          pltpu.VMEM((N_CHUNKS, CHUNK), jnp.int32),
          pltpu.VMEM((N_CHUNKS, CHUNK), jnp.int32),
          pltpu.VMEM((B_PER_W, HIDDEN), jnp.float32),
          pltpu.VMEM((B_PER_W, HIDDEN), jnp.float32),
          pltpu.VMEM((B_PER_W, HIDDEN), jnp.float32),
          pltpu.VMEM((B_PER_W,), jnp.float32),
          pltpu.SemaphoreType.DMA,
      ],
  )
  def kge_kernel(heads_hbm, rels_hbm, tails_hbm, entity_hbm, relation_hbm,
                 out_hbm, idx_h, idx_r, idx_t, h_v, r_v, t_v, out_v, sem):
    wid = lax.axis_index("s") * 2 + lax.axis_index("c")

    pltpu.sync_copy(heads_hbm.at[wid], idx_h)
    pltpu.sync_copy(rels_hbm.at[wid], idx_r)
    pltpu.sync_copy(tails_hbm.at[wid], idx_t)

    for j in range(N_CHUNKS):
      dst = pl.ds(j * CHUNK, CHUNK)
      pltpu.async_copy(entity_hbm.at[idx_h.at[j]], h_v.at[dst], sem)
      pltpu.async_copy(relation_hbm.at[idx_r.at[j]], r_v.at[dst], sem)
      pltpu.async_copy(entity_hbm.at[idx_t.at[j]], t_v.at[dst], sem)
    for j in range(N_CHUNKS):
      dst = pl.ds(j * CHUNK, CHUNK)
      pltpu.make_async_copy(entity_hbm.at[idx_h.at[j]], h_v.at[dst], sem).wait()
      pltpu.make_async_copy(relation_hbm.at[idx_r.at[j]], r_v.at[dst], sem).wait()
      pltpu.make_async_copy(entity_hbm.at[idx_t.at[j]], t_v.at[dst], sem).wait()

    lane = lax.iota(jnp.int32, LANES)

    def group_body(g, _):
      rows = g * LANES + lane
      acc = jnp.zeros((LANES,), jnp.float32)
      for d in range(HIDDEN):
        col = jnp.full((LANES,), d, jnp.int32)
        h = plsc.load_gather(h_v, [rows, col])
        r = plsc.load_gather(r_v, [rows, col])
        t = plsc.load_gather(t_v, [rows, col])
        acc = acc + jnp.abs(h + r - t)
      out_v[pl.ds(g * LANES, LANES)] = GAMMA - acc
      return 0

    lax.fori_loop(0, B_PER_W // LANES, group_body, 0)

    pltpu.sync_copy(out_v, out_hbm.at[wid])

  return kge_kernel


_KERNEL = _make_kernel()


@jax.jit
def kernel(sample, entity_embedding, relation_embedding):
  heads = sample[:, 0].reshape(NUM_WORKERS, N_CHUNKS, CHUNK)
  rels = sample[:, 1].reshape(NUM_WORKERS, N_CHUNKS, CHUNK)
  tails = sample[:, 2].reshape(NUM_WORKERS, N_CHUNKS, CHUNK)
  out = _KERNEL(heads, rels, tails, entity_embedding, relation_embedding)
  return out.reshape(BATCH, 1)
